# Initial kernel scaffold; baseline (speedup 1.0000x reference)
import jax
import jax.numpy as jnp
from jax.experimental import pallas as pl


def kernel(features, rois):
    raise NotImplementedError("write your pallas kernel here")


# speedup vs baseline: 2.7970x; 2.7970x over previous
import jax
import jax.numpy as jnp
from jax import lax
from jax.experimental import pallas as pl
from jax.experimental.pallas import tpu as pltpu
from jax.experimental.pallas import tpu_sc as plsc

H = 64
W = 64
C = 128
GRID = 8
OUT_HW = 7
NPTS = GRID * GRID
OUT_SZ = C * OUT_HW * OUT_HW
SCALE = 0.0625
NW = 32
WPAD = NPTS + 16
CPAD = 176


def _roi_align_avg_sc(featf, coords, n_pad):
    r_per_w = n_pad // NW
    assert r_per_w + 16 <= CPAD and r_per_w % 8 == 0
    mesh = plsc.VectorSubcoreMesh(core_axis_name="c", subcore_axis_name="s")

    def body(featf_hbm, coords_hbm, out_hbm,
             coords_v, idx_v, gbuf, wbuf, sgrid, obuf, gsem, osem):
        cid = lax.axis_index("c")
        sid = lax.axis_index("s")
        wid = sid * 2 + cid
        base = wid * r_per_w
        for k in range(4):
            pltpu.sync_copy(coords_hbm.at[pl.ds(k * n_pad + base, r_per_w)],
                            coords_v.at[pl.ds(k * CPAD, r_per_w)])

        lane = lax.iota(jnp.int32, 16)

        def build_and_fire(rr, nb):
            x1 = coords_v[pl.ds(0 * CPAD + rr, 16)][0] * SCALE
            y1 = coords_v[pl.ds(1 * CPAD + rr, 16)][0] * SCALE
            x2 = coords_v[pl.ds(2 * CPAD + rr, 16)][0] * SCALE
            y2 = coords_v[pl.ds(3 * CPAD + rr, 16)][0] * SCALE
            bin_w = jnp.maximum(x2 - x1, 0.0) * (1.0 / (GRID - 1))
            bin_h = jnp.maximum(y2 - y1, 0.0) * (1.0 / (GRID - 1))
            for q in range(4):
                lin = lane + (q * 16)
                phf = (lin >> 3).astype(jnp.float32)
                pwf = (lin & 7).astype(jnp.float32)
                h = y1 + phf * bin_h
                w = x1 + pwf * bin_w
                hs = jnp.clip(h.astype(jnp.int32), 0, H - 2)
                ws = jnp.clip(w.astype(jnp.int32), 0, W - 2)
                hr = h - hs.astype(jnp.float32)
                wr = w - ws.astype(jnp.float32)
                ok = (h >= 0.0) & (h < float(H)) & (w >= 0.0) & (w < float(W))
                vf = jnp.where(ok, 0.25, 0.0)
                i00 = hs * W + ws
                sl = pl.ds(q * 16, 16)
                idx_v[nb, 0, sl] = i00
                idx_v[nb, 0, pl.ds(64 + q * 16, 16)] = i00 + 1
                idx_v[nb, 1, sl] = i00 + W
                idx_v[nb, 1, pl.ds(64 + q * 16, 16)] = i00 + W + 1
                a0 = vf - vf * hr
                a1 = vf * hr
                wb = nb * (4 * WPAD) + q * 16
                wbuf[pl.ds(wb + 0 * WPAD, 16)] = a0 - a0 * wr
                wbuf[pl.ds(wb + 1 * WPAD, 16)] = a0 * wr
                wbuf[pl.ds(wb + 2 * WPAD, 16)] = a1 - a1 * wr
                wbuf[pl.ds(wb + 3 * WPAD, 16)] = a1 * wr
            for j in range(2):
                pltpu.make_async_copy(featf_hbm.at[idx_v.at[nb, j]],
                                      gbuf.at[nb, pl.ds(j * 128, 128)],
                                      gsem.at[nb]).start()

        def wait_gather(nb):
            for j in range(2):
                pltpu.make_async_copy(featf_hbm.at[idx_v.at[nb, j]],
                                      gbuf.at[nb, pl.ds(j * 128, 128)],
                                      gsem.at[nb]).wait()

        def wait_out(rg, nb):
            pltpu.make_async_copy(obuf.at[nb], out_hbm.at[rg],
                                  osem.at[nb]).wait()

        def combine(nb):
            wb = nb * (4 * WPAD)

            def one_point(p, carry):
                w00 = wbuf[pl.ds(wb + 0 * WPAD + p, 16)][0]
                w01 = wbuf[pl.ds(wb + 1 * WPAD + p, 16)][0]
                w10 = wbuf[pl.ds(wb + 2 * WPAD + p, 16)][0]
                w11 = wbuf[pl.ds(wb + 3 * WPAD + p, 16)][0]
                for cc in range(8):
                    sl = pl.ds(cc * 16, 16)
                    acc = (w00 * gbuf[nb, p, sl]
                           + w01 * gbuf[nb, 64 + p, sl]
                           + w10 * gbuf[nb, 128 + p, sl]
                           + w11 * gbuf[nb, 192 + p, sl])
                    sgrid[p, sl] = acc
                return carry
            lax.fori_loop(0, NPTS, one_point, 0, unroll=False)

        def pool_scatter(nb):
            nbv = jnp.full((16,), nb, jnp.int32)
            lane49 = lane * (OUT_HW * OUT_HW)

            def one_out(op, carry):
                i = op // OUT_HW
                j = op - i * OUT_HW
                b = i * GRID + j
                for cc in range(8):
                    sl = pl.ds(cc * 16, 16)
                    v = (sgrid[b, sl] + sgrid[b + 1, sl]
                         + sgrid[b + GRID, sl] + sgrid[b + GRID + 1, sl])
                    col = lane49 + (cc * 16 * OUT_HW * OUT_HW + op)
                    plsc.store_scatter(obuf, [nbv, col], v)
                return carry
            lax.fori_loop(0, OUT_HW * OUT_HW, one_out, 0, unroll=False)

        build_and_fire(0, 0)
        build_and_fire(1, 1)

        def step(i, carry):
            for nb in range(2):
                r = i * 2 + nb
                wait_gather(nb)
                combine(nb)

                @pl.when(r < r_per_w - 2)
                def _():
                    build_and_fire(r + 2, nb)

                @pl.when(i > 0)
                def _():
                    wait_out(base + r - 2, nb)

                pool_scatter(nb)
                pltpu.make_async_copy(obuf.at[nb], out_hbm.at[base + r],
                                      osem.at[nb]).start()
            return carry

        lax.fori_loop(0, r_per_w // 2, step, 0, unroll=False)
        for nb in range(2):
            wait_out(base + r_per_w - 2 + nb, nb)

    call = pl.kernel(
        body,
        out_type=jax.ShapeDtypeStruct((n_pad, OUT_SZ), jnp.float32),
        mesh=mesh,
        compiler_params=pltpu.CompilerParams(needs_layout_passes=False),
        scratch_types=[
            pltpu.VMEM((4 * CPAD,), jnp.float32),
            pltpu.VMEM((2, 2, 128), jnp.int32),
            pltpu.VMEM((2, 256, C), jnp.float32),
            pltpu.VMEM((2 * 4 * WPAD,), jnp.float32),
            pltpu.VMEM((NPTS, C), jnp.float32),
            pltpu.VMEM((2, OUT_SZ), jnp.float32),
            pltpu.SemaphoreType.DMA((2,)),
            pltpu.SemaphoreType.DMA((2,)),
        ],
    )
    return call(featf, coords)


@jax.jit
def kernel(features, rois):
    n = rois.shape[0]
    n_pad = -(-n // (NW * 8)) * (NW * 8)
    featf = features.transpose(0, 2, 3, 1).reshape(H * W, C)
    coords = jnp.zeros((4, n_pad), jnp.float32)
    coords = coords.at[:, :n].set(rois[:, 1:5].T)
    out = _roi_align_avg_sc(featf, coords.reshape(-1), n_pad)
    return out[:n].reshape(n, C, OUT_HW, OUT_HW)


# speedup vs baseline: 4.4948x; 1.6070x over previous
import jax
import jax.numpy as jnp
from jax import lax
from jax.experimental import pallas as pl
from jax.experimental.pallas import tpu as pltpu
from jax.experimental.pallas import tpu_sc as plsc

H = 64
W = 64
C = 128
GRID = 8
OUT_HW = 7
NPTS = GRID * GRID
NOUT = OUT_HW * OUT_HW
NOUTP = 56
SCALE = 0.0625
NW = 32
WPAD = NPTS + 16


def _roi_align_avg_sc(featf, coords, n, r_per_w, cpad):
    csz = ((r_per_w + 14) // 8) * 8
    n_host = coords.shape[0] // 4
    mesh = plsc.VectorSubcoreMesh(core_axis_name="c", subcore_axis_name="s")

    def body(featf_hbm, coords_hbm, out_hbm,
             coords_v, idx_v, gbuf, wbuf, sgrid, obuf, oidx, gsem, osem):
        cid = lax.axis_index("c")
        sid = lax.axis_index("s")
        wid = sid * 2 + cid
        base = wid * r_per_w
        abase = (base // 8) * 8
        d = base - abase
        for k in range(4):
            pltpu.sync_copy(coords_hbm.at[pl.ds(k * n_host + abase, csz)],
                            coords_v.at[pl.ds(k * cpad, csz)])

        lane = lax.iota(jnp.int32, 16)

        def build_and_fire(rr, nb):
            x1 = coords_v[pl.ds(0 * cpad + d + rr, 16)][0] * SCALE
            y1 = coords_v[pl.ds(1 * cpad + d + rr, 16)][0] * SCALE
            x2 = coords_v[pl.ds(2 * cpad + d + rr, 16)][0] * SCALE
            y2 = coords_v[pl.ds(3 * cpad + d + rr, 16)][0] * SCALE
            bin_w = jnp.maximum(x2 - x1, 0.0) * (1.0 / (GRID - 1))
            bin_h = jnp.maximum(y2 - y1, 0.0) * (1.0 / (GRID - 1))
            for q in range(4):
                lin = lane + (q * 16)
                phf = (lin >> 3).astype(jnp.float32)
                pwf = (lin & 7).astype(jnp.float32)
                h = y1 + phf * bin_h
                w = x1 + pwf * bin_w
                hs = jnp.clip(h.astype(jnp.int32), 0, H - 2)
                ws = jnp.clip(w.astype(jnp.int32), 0, W - 2)
                hr = h - hs.astype(jnp.float32)
                wr = w - ws.astype(jnp.float32)
                ok = (h >= 0.0) & (h < float(H)) & (w >= 0.0) & (w < float(W))
                vf = jnp.where(ok, 0.25, 0.0)
                i00 = hs * W + ws
                sl = pl.ds(q * 16, 16)
                idx_v[nb, 0, sl] = i00
                idx_v[nb, 0, pl.ds(64 + q * 16, 16)] = i00 + 1
                idx_v[nb, 1, sl] = i00 + W
                idx_v[nb, 1, pl.ds(64 + q * 16, 16)] = i00 + W + 1
                a0 = vf - vf * hr
                a1 = vf * hr
                wb = nb * (4 * WPAD) + q * 16
                wbuf[pl.ds(wb + 0 * WPAD, 16)] = a0 - a0 * wr
                wbuf[pl.ds(wb + 1 * WPAD, 16)] = a0 * wr
                wbuf[pl.ds(wb + 2 * WPAD, 16)] = a1 - a1 * wr
                wbuf[pl.ds(wb + 3 * WPAD, 16)] = a1 * wr
            for j in range(2):
                pltpu.make_async_copy(featf_hbm.at[idx_v.at[nb, j]],
                                      gbuf.at[nb, pl.ds(j * 128, 128)],
                                      gsem.at[nb]).start()

        def wait_gather(nb):
            for j in range(2):
                pltpu.make_async_copy(featf_hbm.at[idx_v.at[nb, j]],
                                      gbuf.at[nb, pl.ds(j * 128, 128)],
                                      gsem.at[nb]).wait()

        def out_copy(nb):
            return pltpu.make_async_copy(obuf.at[nb],
                                         out_hbm.at[oidx.at[nb]],
                                         osem.at[nb])

        def combine(nb):
            wb = nb * (4 * WPAD)

            def one_point(p, carry):
                w00 = wbuf[pl.ds(wb + 0 * WPAD + p, 16)][0]
                w01 = wbuf[pl.ds(wb + 1 * WPAD + p, 16)][0]
                w10 = wbuf[pl.ds(wb + 2 * WPAD + p, 16)][0]
                w11 = wbuf[pl.ds(wb + 3 * WPAD + p, 16)][0]
                for cc in range(8):
                    sl = pl.ds(cc * 16, 16)
                    acc = (w00 * gbuf[nb, p, sl]
                           + w01 * gbuf[nb, 64 + p, sl]
                           + w10 * gbuf[nb, 128 + p, sl]
                           + w11 * gbuf[nb, 192 + p, sl])
                    sgrid[p, sl] = acc
                return carry
            lax.fori_loop(0, NPTS, one_point, 0, unroll=False)

        def pool(nb, rg):
            rc = jnp.minimum(rg, n - 1)
            for st in (0, 16, 32, 40):
                opc = jnp.minimum(lane + st, NOUT - 1)
                oidx[nb, pl.ds(st, 16)] = opc * n + rc

            def one_out(op, carry):
                op_c = jnp.minimum(op, NOUT - 1)
                i = op_c // OUT_HW
                j = op_c - i * OUT_HW
                b = i * GRID + j
                for cc in range(8):
                    sl = pl.ds(cc * 16, 16)
                    v = (sgrid[b, sl] + sgrid[b + 1, sl]
                         + sgrid[b + GRID, sl] + sgrid[b + GRID + 1, sl])
                    obuf[nb, op, sl] = v
                return carry
            lax.fori_loop(0, NOUTP, one_out, 0, unroll=False)

        build_and_fire(0, 0)
        build_and_fire(1, 1)

        def step(i, carry):
            for nb in range(2):
                r = i * 2 + nb
                rg = base + r
                wait_gather(nb)
                combine(nb)

                @pl.when(r < r_per_w - 2)
                def _():
                    build_and_fire(r + 2, nb)

                @pl.when((i > 0) & (rg - 2 < n))
                def _():
                    out_copy(nb).wait()

                pool(nb, rg)

                @pl.when(rg < n)
                def _():
                    out_copy(nb).start()
            return carry

        lax.fori_loop(0, r_per_w // 2, step, 0, unroll=False)
        for nb in range(2):
            rg = base + r_per_w - 2 + nb

            @pl.when(rg < n)
            def _():
                out_copy(nb).wait()

    call = pl.kernel(
        body,
        out_type=jax.ShapeDtypeStruct((NOUT * n, C), jnp.float32),
        mesh=mesh,
        compiler_params=pltpu.CompilerParams(needs_layout_passes=False),
        scratch_types=[
            pltpu.VMEM((4 * cpad,), jnp.float32),
            pltpu.VMEM((2, 2, 128), jnp.int32),
            pltpu.VMEM((2, 256, C), jnp.float32),
            pltpu.VMEM((2 * 4 * WPAD,), jnp.float32),
            pltpu.VMEM((NPTS, C), jnp.float32),
            pltpu.VMEM((2, NOUTP, C), jnp.float32),
            pltpu.VMEM((2, NOUTP), jnp.int32),
            pltpu.SemaphoreType.DMA((2,)),
            pltpu.SemaphoreType.DMA((2,)),
        ],
    )
    return call(featf, coords)


@jax.jit
def kernel(features, rois):
    n = rois.shape[0]
    r_per_w = -(-n // NW)
    csz = ((r_per_w + 14) // 8) * 8
    cpad = csz + 16
    n_host = ((NW * r_per_w + 15) // 8) * 8
    featf = features.transpose(0, 2, 3, 1).reshape(H * W, C)
    coords = jnp.zeros((4, n_host), jnp.float32)
    coords = coords.at[:, :n].set(rois[:, 1:5].T)
    out = _roi_align_avg_sc(featf, coords.reshape(-1), n, r_per_w, cpad)
    return out.reshape(OUT_HW, OUT_HW, n, C).transpose(2, 3, 0, 1)
